# Initial kernel scaffold; baseline (speedup 1.0000x reference)
import jax
import jax.numpy as jnp
from jax.experimental import pallas as pl


def kernel(x, edge_index, conv1d_w, conv1d_b, W1_self, W1_neigh, b1, W2_self, W2_neigh, b2, fc1_w, fc1_b, fc2_w, fc2_b, fc3_w, fc3_b):
    raise NotImplementedError("write your pallas kernel here")


# speedup vs baseline: 6.8989x; 6.8989x over previous
import functools

import jax
import jax.numpy as jnp
from jax import lax
from jax.experimental import pallas as pl
from jax.experimental.pallas import tpu as pltpu
from jax.experimental.pallas import tpu_sc as plsc

N = 10000
E = 320000
D = 128
KS = 10
ST = 8
CONV_OUT = 15
F1 = 16
H = 128
H2 = 64

NC = 2
NS = 16
NW = NC * NS
CH = 128
K = 8
N_PAD = 10240
E_PAD = NW * 80 * CH
EW = E_PAD // NW
SUPERS = EW // (K * CH)
ROWS_PER_TILE = N_PAD // NS


def _make_seg_sum(F):
  mesh = plsc.VectorSubcoreMesh(core_axis_name="c", subcore_axis_name="s")

  @functools.partial(
      pl.kernel,
      out_type=jax.ShapeDtypeStruct((2 * N_PAD, F), jnp.float32),
      mesh=mesh,
      scratch_types=[
          pltpu.VMEM((K, CH), jnp.int32),
          pltpu.VMEM((K, CH), jnp.int32),
          pltpu.VMEM((K, CH, F), jnp.float32),
          pltpu.VMEM_SHARED((N_PAD, F), jnp.float32),
          pltpu.SemaphoreType.DMA,
      ],
      compiler_params=pltpu.CompilerParams(use_tc_tiling_on_sc=False),
  )
  def seg(feat_hbm, src_hbm, dst_hbm, out_hbm, srcb, dstb, rows, acc, sem):
    c = lax.axis_index("c")
    s = lax.axis_index("s")
    w = c * NS + s

    def zrow(r, carry):
      for j in range(F // 16):
        rows[0, r, pl.ds(16 * j, 16)] = jnp.zeros((16,), jnp.float32)
      return carry

    lax.fori_loop(0, CH, zrow, 0)
    zbase = s * ROWS_PER_TILE
    for k2 in range(ROWS_PER_TILE // CH):
      pltpu.sync_copy(rows.at[0], acc.at[pl.ds(zbase + k2 * CH, CH)])
    plsc.subcore_barrier()

    nchunks = EW // CH

    def body(t, carry):
      r0 = w * nchunks + t * K
      pltpu.sync_copy(src_hbm.at[pl.ds(r0, K)], srcb)
      pltpu.sync_copy(dst_hbm.at[pl.ds(r0, K)], dstb)
      cps = [
          pltpu.async_copy(feat_hbm.at[srcb.at[b]], rows.at[b], sem)
          for b in range(K)
      ]
      for cp in cps:
        cp.wait()
      for b in range(K):
        pltpu.sync_copy(rows.at[b], acc.at[dstb.at[b]], add=True)
      return carry

    lax.fori_loop(0, SUPERS, body, 0)
    plsc.subcore_barrier()

    for k2 in range(ROWS_PER_TILE // CH):
      r = s * ROWS_PER_TILE + k2 * CH
      pltpu.sync_copy(acc.at[pl.ds(r, CH)], out_hbm.at[pl.ds(c * N_PAD + r, CH)])

  return seg


_seg_sum_16 = _make_seg_sum(F1)
_seg_sum_64 = _make_seg_sum(H2)


BLK = 1024
BLK_C = 1000


def _conv_body(x_ref, c_ref, be_ref, o_ref):
  i = pl.program_id(0)
  y = jnp.dot(x_ref[...], c_ref[...], preferred_element_type=jnp.float32)
  y = jnp.maximum(y + be_ref[...], 0.0)
  col = lax.broadcasted_iota(jnp.int32, y.shape, 1)
  y = jnp.where(col == CONV_OUT, 1.0, y)
  row = lax.broadcasted_iota(jnp.int32, y.shape, 0) + i * BLK
  o_ref[...] = jnp.where(row < N, y, 0.0)


def _mid_body(h0_ref, agg_ref, w1s_ref, w1n_ref, b1_ref, w2st_ref, w2nt_ref,
              p1_ref, s1_ref, deg_ref):
  i = pl.program_id(0)
  a = agg_ref[0] + agg_ref[1]
  deg = jnp.maximum(a[:, CONV_OUT:CONV_OUT + 1], 1.0)
  aggm = a / deg
  h1 = jnp.dot(h0_ref[...], w1s_ref[...], preferred_element_type=jnp.float32)
  h1 += jnp.dot(aggm, w1n_ref[...], preferred_element_type=jnp.float32)
  h1 = jnp.maximum(h1 + b1_ref[...], 0.0)
  row = lax.broadcasted_iota(jnp.int32, (BLK, H2), 0) + i * BLK
  ok = row < N
  p1 = jnp.dot(h1, w2nt_ref[...], preferred_element_type=jnp.float32)
  s1 = jnp.dot(h1, w2st_ref[...], preferred_element_type=jnp.float32)
  p1_ref[...] = jnp.where(ok, p1, 0.0)
  s1_ref[...] = jnp.where(ok, s1, 0.0)
  deg_ref[...] = deg


def _final_body(s1_ref, agg2_ref, deg_ref, b2_ref, f1_ref, f1b_ref, f2_ref,
                f2b_ref, f3_ref, f3b_ref, o_ref, acc_ref):
  i = pl.program_id(0)
  a = (agg2_ref[0] + agg2_ref[1]) / deg_ref[...]
  h2 = jnp.maximum(s1_ref[...] + a + b2_ref[...], 0.0)
  colsum = jnp.sum(h2, axis=0, keepdims=True)

  @pl.when(i == 0)
  def _():
    acc_ref[...] = colsum

  @pl.when(i > 0)
  def _():
    acc_ref[...] = acc_ref[...] + colsum

  @pl.when(i == pl.num_programs(0) - 1)
  def _():
    hg = acc_ref[...] * (1.0 / N)
    t = jnp.maximum(
        jnp.dot(hg, f1_ref[...], preferred_element_type=jnp.float32)
        + f1b_ref[...], 0.0)
    t = jnp.maximum(
        jnp.dot(t, f2_ref[...], preferred_element_type=jnp.float32)
        + f2b_ref[...], 0.0)
    o_ref[...] = (
        jnp.dot(t, f3_ref[...], preferred_element_type=jnp.float32)
        + f3b_ref[...])


def kernel(x, edge_index, conv1d_w, conv1d_b, W1_self, W1_neigh, b1,
           W2_self, W2_neigh, b2, fc1_w, fc1_b, fc2_w, fc2_b, fc3_w, fc3_b):
  f32 = jnp.float32

  x_pad = jnp.pad(x, ((0, N_PAD - N), (0, 0)))
  src = edge_index[0]
  dst = edge_index[1]
  pad_e = jnp.full((E_PAD - E,), N, jnp.int32)
  src2d = jnp.concatenate([src, pad_e]).reshape(E_PAD // CH, CH)
  dst2d = jnp.concatenate([dst, pad_e]).reshape(E_PAD // CH, CH)

  w = conv1d_w.reshape(KS)
  tt = jnp.arange(CONV_OUT, dtype=jnp.int32)
  kk = jnp.arange(KS, dtype=jnp.int32)
  rows_idx = (ST * tt[:, None] + kk[None, :]).reshape(-1)
  cols_idx = jnp.broadcast_to(tt[:, None], (CONV_OUT, KS)).reshape(-1)
  C = jnp.zeros((D, F1), f32).at[rows_idx, cols_idx].set(
      jnp.broadcast_to(w[None, :], (CONV_OUT, KS)).reshape(-1))
  b_ext = jnp.zeros((1, F1), f32).at[0, :CONV_OUT].set(conv1d_b[0])

  W1s = jnp.pad(W1_self.T, ((0, 1), (0, 0)))
  W1n = jnp.pad(W1_neigh.T, ((0, 1), (0, 0)))
  b1r = b1.reshape(1, H)
  W2sT = W2_self.T
  W2nT = W2_neigh.T
  b2r = b2.reshape(1, H2)
  f1 = fc1_w.T
  f1b = fc1_b.reshape(1, -1)
  f2 = fc2_w.T
  f2b = fc2_b.reshape(1, -1)
  f3 = fc3_w.T
  f3b = fc3_b.reshape(1, -1)

  h0p = pl.pallas_call(
      _conv_body,
      grid=(N_PAD // BLK,),
      in_specs=[
          pl.BlockSpec((BLK, D), lambda i: (i, 0)),
          pl.BlockSpec((D, F1), lambda i: (0, 0)),
          pl.BlockSpec((1, F1), lambda i: (0, 0)),
      ],
      out_specs=pl.BlockSpec((BLK, F1), lambda i: (i, 0)),
      out_shape=jax.ShapeDtypeStruct((N_PAD, F1), f32),
  )(x_pad, C, b_ext)

  agg1 = _seg_sum_16(h0p, src2d, dst2d).reshape(2, N_PAD, F1)

  p1, s1, deg = pl.pallas_call(
      _mid_body,
      grid=(N_PAD // BLK,),
      in_specs=[
          pl.BlockSpec((BLK, F1), lambda i: (i, 0)),
          pl.BlockSpec((2, BLK, F1), lambda i: (0, i, 0)),
          pl.BlockSpec((F1, H), lambda i: (0, 0)),
          pl.BlockSpec((F1, H), lambda i: (0, 0)),
          pl.BlockSpec((1, H), lambda i: (0, 0)),
          pl.BlockSpec((H, H2), lambda i: (0, 0)),
          pl.BlockSpec((H, H2), lambda i: (0, 0)),
      ],
      out_specs=[
          pl.BlockSpec((BLK, H2), lambda i: (i, 0)),
          pl.BlockSpec((BLK, H2), lambda i: (i, 0)),
          pl.BlockSpec((BLK, 1), lambda i: (i, 0)),
      ],
      out_shape=[
          jax.ShapeDtypeStruct((N_PAD, H2), f32),
          jax.ShapeDtypeStruct((N_PAD, H2), f32),
          jax.ShapeDtypeStruct((N_PAD, 1), f32),
      ],
  )(h0p, agg1, W1s, W1n, b1r, W2sT, W2nT)

  agg2 = _seg_sum_64(p1, src2d, dst2d).reshape(2, N_PAD, H2)

  out = pl.pallas_call(
      _final_body,
      grid=(N // BLK_C,),
      in_specs=[
          pl.BlockSpec((BLK_C, H2), lambda i: (i, 0)),
          pl.BlockSpec((2, BLK_C, H2), lambda i: (0, i, 0)),
          pl.BlockSpec((BLK_C, 1), lambda i: (i, 0)),
          pl.BlockSpec((1, H2), lambda i: (0, 0)),
          pl.BlockSpec((H2, H // 4), lambda i: (0, 0)),
          pl.BlockSpec((1, H // 4), lambda i: (0, 0)),
          pl.BlockSpec((H // 4, H // 8), lambda i: (0, 0)),
          pl.BlockSpec((1, H // 8), lambda i: (0, 0)),
          pl.BlockSpec((H // 8, 1), lambda i: (0, 0)),
          pl.BlockSpec((1, 1), lambda i: (0, 0)),
      ],
      out_specs=pl.BlockSpec((1, 1), lambda i: (0, 0)),
      out_shape=jax.ShapeDtypeStruct((1, 1), f32),
      scratch_shapes=[pltpu.VMEM((1, H2), f32)],
  )(s1, agg2, deg, b2r, f1, f1b, f2, f2b, f3, f3b)

  return out
